# Initial kernel scaffold; baseline (speedup 1.0000x reference)
import jax
import jax.numpy as jnp
from jax.experimental import pallas as pl


def kernel(x, edge_index, edge_weights, feature_mask, W0, b0, W1, b1):
    raise NotImplementedError("write your pallas kernel here")


# speedup vs baseline: 3.0462x; 3.0462x over previous
import functools

import jax
import jax.numpy as jnp
from jax import lax
from jax.experimental import pallas as pl
from jax.experimental.pallas import tpu as pltpu
from jax.experimental.pallas import tpu_sc as plsc

N = 10000
E = 320000
F = 128
FH = 64
KHOPS = 3

NC = 2
NS = 16
CH = 80
EPT = E // NS
NCH = EPT // CH
EPW = E // (NC * NS)
NCHW = EPW // CH
NR = 10240
RPT = NR // NS
ZROWS = 128
DEG_PAD = 10240

_mesh = plsc.VectorSubcoreMesh(core_axis_name="c", subcore_axis_name="s")

_GATHER_1D = lax.GatherDimensionNumbers(
    offset_dims=(), collapsed_slice_dims=(0,), start_index_map=(0,))


def _bcast_lane(v16, lane):
    idx = jnp.full((16, 1), lane, _i32)
    return lax.gather(v16, idx, _GATHER_1D, (1,),
                      mode=lax.GatherScatterMode.PROMISE_IN_BOUNDS)

_f32 = jnp.float32
_i32 = jnp.int32


def _fast_rsqrt(v):
    i = lax.bitcast_convert_type(v, _i32)
    y = lax.bitcast_convert_type(jnp.int32(0x5F3759DF) - (i >> 1), _f32)
    for _ in range(3):
        y = y * (1.5 - 0.5 * v * y * y)
    return y


@functools.partial(
    pl.kernel,
    out_type=jax.ShapeDtypeStruct((E,), _f32),
    mesh=_mesh,
    scratch_types=[
        pltpu.VMEM((640,), _f32),
        pltpu.VMEM((CH,), _i32),
        pltpu.VMEM((CH,), _i32),
        pltpu.VMEM((CH,), _f32),
        pltpu.VMEM((CH,), _f32),
        pltpu.VMEM((N,), _f32),
        pltpu.VMEM_SHARED((DEG_PAD,), _f32),
    ],
    compiler_params=pltpu.CompilerParams(needs_layout_passes=False),
    name="tag_prep_norm",
)
def _prep(src_hbm, dst_hbm, ew_hbm, norm_hbm,
          zb, idx_d, idx_s, ewb, nrmb, dinv, deg_sh):
    c = lax.axis_index("c")
    s = lax.axis_index("s")

    def zb_body(i, carry):
        zb[pl.ds(i * 16, 16)] = jnp.zeros((16,), _f32)
        return carry

    lax.fori_loop(0, 640 // 16, zb_body, 0)
    pltpu.sync_copy(zb, deg_sh.at[pl.ds(s * 640, 640)])
    plsc.subcore_barrier()

    def deg_body(ci, carry):
        base = s * EPT + ci * CH
        pltpu.sync_copy(dst_hbm.at[pl.ds(base, CH)], idx_d)
        pltpu.sync_copy(ew_hbm.at[pl.ds(base, CH)], ewb)
        pltpu.sync_copy(ewb, deg_sh.at[idx_d], add=True)
        return carry

    lax.fori_loop(0, NCH, deg_body, 0)
    plsc.subcore_barrier()

    pltpu.sync_copy(deg_sh.at[pl.ds(0, N)], dinv)

    def dinv_body(i, carry):
        v = dinv[pl.ds(i * 16, 16)]
        r = _fast_rsqrt(v)
        dinv[pl.ds(i * 16, 16)] = jnp.where(v > 0.0, r, 0.0)
        return carry

    lax.fori_loop(0, N // 16, dinv_body, 0)

    wid = c * NS + s

    def norm_body(ci, carry):
        base = wid * EPW + ci * CH
        pltpu.sync_copy(src_hbm.at[pl.ds(base, CH)], idx_s)
        pltpu.sync_copy(dst_hbm.at[pl.ds(base, CH)], idx_d)
        pltpu.sync_copy(ew_hbm.at[pl.ds(base, CH)], ewb)
        for g in range(CH // 16):
            sl = pl.ds(g * 16, 16)
            a = plsc.load_gather(dinv, [idx_s[sl]])
            b = plsc.load_gather(dinv, [idx_d[sl]])
            nrmb[sl] = a * ewb[sl] * b
        pltpu.sync_copy(nrmb, norm_hbm.at[pl.ds(base, CH)])
        return carry

    lax.fori_loop(0, NCHW, norm_body, 0)


@functools.partial(
    pl.kernel,
    out_type=[jax.ShapeDtypeStruct((NR, FH), _f32) for _ in range(2 * KHOPS)],
    mesh=_mesh,
    scratch_types=[
        pltpu.VMEM((ZROWS, FH), _f32),
        pltpu.VMEM((CH,), _i32),
        pltpu.VMEM((CH,), _i32),
        pltpu.VMEM((CH,), _f32),
        pltpu.VMEM((CH, FH), _f32),
        pltpu.VMEM_SHARED((NR, FH), _f32),
        pltpu.SemaphoreType.DMA,
    ],
    compiler_params=pltpu.CompilerParams(needs_layout_passes=False,
                                         use_tc_tiling_on_sc=False),
    name="tag_propagate",
)
def _layer(h0_hbm, h1_hbm, src_hbm, dst_hbm, norm_hbm,
           o10, o11, o20, o21, o30, o31,
           zb, idx_s, idx_d, nrm, rows, acc, gsem):
    c = lax.axis_index("c")
    s = lax.axis_index("s")

    def zb_body(i, carry):
        for j in range(FH // 16):
            zb[i, pl.ds(j * 16, 16)] = jnp.zeros((16,), _f32)
        return carry

    lax.fori_loop(0, ZROWS, zb_body, 0)

    srcs = [(h0_hbm, h1_hbm), (o10, o11), (o20, o21)]
    outs = [(o10, o11), (o20, o21), (o30, o31)]
    for r in range(KHOPS):
        for k2 in range(RPT // ZROWS):
            pltpu.sync_copy(zb, acc.at[pl.ds(s * RPT + k2 * ZROWS, ZROWS)])
        plsc.subcore_barrier()

        g0, g1 = srcs[r]

        def chunk_body(ci, carry, g0=g0, g1=g1):
            base = s * EPT + ci * CH
            pltpu.sync_copy(src_hbm.at[pl.ds(base, CH)], idx_s)
            pltpu.sync_copy(dst_hbm.at[pl.ds(base, CH)], idx_d)
            pltpu.sync_copy(norm_hbm.at[pl.ds(base, CH)], nrm)

            @pl.when(c == 0)
            def _():
                pltpu.async_copy(g0.at[idx_s], rows, gsem).wait()

            @pl.when(c == 1)
            def _():
                pltpu.async_copy(g1.at[idx_s], rows, gsem).wait()

            for g in range(CH // 16):
                n16 = nrm[pl.ds(g * 16, 16)]
                for lane in range(16):
                    e = g * 16 + lane
                    bb = _bcast_lane(n16, lane)
                    for j in range(FH // 16):
                        sl = pl.ds(j * 16, 16)
                        rows[e, sl] = rows[e, sl] * bb
            pltpu.sync_copy(rows, acc.at[idx_d], add=True)
            return carry

        lax.fori_loop(0, NCH, chunk_body, 0)
        plsc.subcore_barrier()

        oo0, oo1 = outs[r]
        row0 = s * RPT

        @pl.when(c == 0)
        def _(oo0=oo0):
            pltpu.sync_copy(acc.at[pl.ds(row0, RPT)], oo0.at[pl.ds(row0, RPT)])

        @pl.when(c == 1)
        def _(oo1=oo1):
            pltpu.sync_copy(acc.at[pl.ds(row0, RPT)], oo1.at[pl.ds(row0, RPT)])

        plsc.subcore_barrier()


BM = 2000
_NBM = N // BM


def _mm_acc(parts, w_ref, b_ref):
    acc = b_ref[...]
    for i, p in enumerate(parts):
        acc = acc + jnp.dot(p[...], w_ref[i * FH:(i + 1) * FH, :],
                            preferred_element_type=_f32)
    return acc


def _mm_mid_body(p0, p1, p2, p3, p4, p5, p6, p7, w_ref, b_ref, y0, y1):
    acc = _mm_acc([p0, p1, p2, p3, p4, p5, p6, p7], w_ref, b_ref)
    acc = jnp.where(acc >= 0.0, acc, 0.01 * acc)
    y0[...] = acc[:, :FH]
    y1[...] = acc[:, FH:]


def _mm_out_body(p0, p1, p2, p3, p4, p5, p6, p7, w_ref, b_ref, m_ref, out):
    acc = _mm_acc([p0, p1, p2, p3, p4, p5, p6, p7], w_ref, b_ref)
    out[...] = acc * m_ref[...]


_part_spec = pl.BlockSpec((BM, FH), lambda i: (i, 0))
_w_spec = pl.BlockSpec(((KHOPS + 1) * F, F), lambda i: (0, 0))
_b_spec = pl.BlockSpec((1, F), lambda i: (0, 0))

_mm_mid = pl.pallas_call(
    _mm_mid_body,
    grid=(_NBM,),
    in_specs=[_part_spec] * 8 + [_w_spec, _b_spec],
    out_specs=[pl.BlockSpec((BM, FH), lambda i: (i, 0))] * 2,
    out_shape=[jax.ShapeDtypeStruct((N, FH), _f32)] * 2,
)

_mm_out = pl.pallas_call(
    _mm_out_body,
    grid=(_NBM,),
    in_specs=[_part_spec] * 8 + [_w_spec, _b_spec,
                                 pl.BlockSpec((BM, 1), lambda i: (i, 0))],
    out_specs=pl.BlockSpec((BM, F), lambda i: (i, 0)),
    out_shape=jax.ShapeDtypeStruct((N, F), _f32),
)


def kernel(x, edge_index, edge_weights, feature_mask, W0, b0, W1, b1):
    src = edge_index[0]
    dst = edge_index[1]
    norm = _prep(src, dst, edge_weights)

    x0 = x[:, :FH]
    x1 = x[:, FH:]
    h10, h11, h20, h21, h30, h31 = _layer(x0, x1, src, dst, norm)
    y0, y1 = _mm_mid(x0, x1, h10, h11, h20, h21, h30, h31,
                     W0, b0.reshape(1, F))
    g10, g11, g20, g21, g30, g31 = _layer(y0, y1, src, dst, norm)
    out = _mm_out(y0, y1, g10, g11, g20, g21, g30, g31,
                  W1, b1.reshape(1, F), feature_mask[:, None])
    return out


# speedup vs baseline: 8.2987x; 2.7242x over previous
import functools

import jax
import jax.numpy as jnp
from jax import lax
from jax.experimental import pallas as pl
from jax.experimental.pallas import tpu as pltpu
from jax.experimental.pallas import tpu_sc as plsc

N = 10000
E = 320000
F = 128
FH = 64
KHOPS = 3

NC = 2
NS = 16
CH = 80
EPT = E // NS
NCH = EPT // CH
EPW = E // (NC * NS)
NCHW = EPW // CH
NR = 10240
RPT = NR // NS
ZROWS = 128
DEG_PAD = 10240

_mesh = plsc.VectorSubcoreMesh(core_axis_name="c", subcore_axis_name="s")

_GATHER_1D = lax.GatherDimensionNumbers(
    offset_dims=(), collapsed_slice_dims=(0,), start_index_map=(0,))


def _bcast_lane(v16, lane):
    idx = jnp.full((16, 1), lane, _i32)
    return lax.gather(v16, idx, _GATHER_1D, (1,),
                      mode=lax.GatherScatterMode.PROMISE_IN_BOUNDS)

_f32 = jnp.float32
_i32 = jnp.int32


def _fast_rsqrt(v):
    i = lax.bitcast_convert_type(v, _i32)
    y = lax.bitcast_convert_type(jnp.int32(0x5F3759DF) - (i >> 1), _f32)
    for _ in range(3):
        y = y * (1.5 - 0.5 * v * y * y)
    return y


@functools.partial(
    pl.kernel,
    out_type=jax.ShapeDtypeStruct((E,), _f32),
    mesh=_mesh,
    scratch_types=[
        pltpu.VMEM((640,), _f32),
        pltpu.VMEM((CH,), _i32),
        pltpu.VMEM((CH,), _i32),
        pltpu.VMEM((CH,), _f32),
        pltpu.VMEM((CH,), _f32),
        pltpu.VMEM((N,), _f32),
        pltpu.VMEM_SHARED((DEG_PAD,), _f32),
    ],
    compiler_params=pltpu.CompilerParams(needs_layout_passes=False),
    name="tag_prep_norm",
)
def _prep(src_hbm, dst_hbm, ew_hbm, norm_hbm,
          zb, idx_d, idx_s, ewb, nrmb, dinv, deg_sh):
    c = lax.axis_index("c")
    s = lax.axis_index("s")

    def zb_body(i, carry):
        zb[pl.ds(i * 16, 16)] = jnp.zeros((16,), _f32)
        return carry

    lax.fori_loop(0, 640 // 16, zb_body, 0)
    pltpu.sync_copy(zb, deg_sh.at[pl.ds(s * 640, 640)])
    plsc.subcore_barrier()

    def deg_body(ci, carry):
        base = s * EPT + ci * CH
        pltpu.sync_copy(dst_hbm.at[pl.ds(base, CH)], idx_d)
        pltpu.sync_copy(ew_hbm.at[pl.ds(base, CH)], ewb)
        pltpu.sync_copy(ewb, deg_sh.at[idx_d], add=True)
        return carry

    lax.fori_loop(0, NCH, deg_body, 0)
    plsc.subcore_barrier()

    pltpu.sync_copy(deg_sh.at[pl.ds(0, N)], dinv)

    def dinv_body(i, carry):
        v = dinv[pl.ds(i * 16, 16)]
        r = _fast_rsqrt(v)
        dinv[pl.ds(i * 16, 16)] = jnp.where(v > 0.0, r, 0.0)
        return carry

    lax.fori_loop(0, N // 16, dinv_body, 0)

    wid = c * NS + s

    def norm_body(ci, carry):
        base = wid * EPW + ci * CH
        pltpu.sync_copy(src_hbm.at[pl.ds(base, CH)], idx_s)
        pltpu.sync_copy(dst_hbm.at[pl.ds(base, CH)], idx_d)
        pltpu.sync_copy(ew_hbm.at[pl.ds(base, CH)], ewb)
        for g in range(CH // 16):
            sl = pl.ds(g * 16, 16)
            a = plsc.load_gather(dinv, [idx_s[sl]])
            b = plsc.load_gather(dinv, [idx_d[sl]])
            nrmb[sl] = a * ewb[sl] * b
        pltpu.sync_copy(nrmb, norm_hbm.at[pl.ds(base, CH)])
        return carry

    lax.fori_loop(0, NCHW, norm_body, 0)


NRING = 4
NGRP = NCH // NRING
NEPI = NCH - NGRP * NRING
NCHT = EPT // CH


def _scale_rows(rows, nrm2, ci):
    for g in range(CH // 16):
        n16 = nrm2[ci, pl.ds(g * 16, 16)]
        for lane in range(16):
            e = g * 16 + lane
            bb = _bcast_lane(n16, lane)
            for j in range(FH // 16):
                sl = pl.ds(j * 16, 16)
                rows[e, sl] = rows[e, sl] * bb


@functools.partial(
    pl.kernel,
    out_type=[jax.ShapeDtypeStruct((NR, FH), _f32) for _ in range(2 * KHOPS)],
    mesh=_mesh,
    scratch_types=[
        pltpu.VMEM((ZROWS, FH), _f32),
        pltpu.VMEM((NCHT, CH), _i32),
        pltpu.VMEM((NCHT, CH), _i32),
        pltpu.VMEM((NCHT, CH), _f32),
        [pltpu.VMEM((CH, FH), _f32) for _ in range(NRING)],
        [pltpu.SemaphoreType.DMA for _ in range(NRING)],
        [pltpu.SemaphoreType.DMA for _ in range(NRING)],
        pltpu.VMEM_SHARED((NR, FH), _f32),
    ],
    compiler_params=pltpu.CompilerParams(needs_layout_passes=False,
                                         use_tc_tiling_on_sc=False),
    name="tag_propagate",
)
def _layer(h0_hbm, h1_hbm, src_hbm, dst_hbm, norm_hbm,
           o10, o11, o20, o21, o30, o31,
           zb, idx_s2, idx_d2, nrm2, rows, gsem, ssem, acc):
    c = lax.axis_index("c")
    s = lax.axis_index("s")

    def zb_body(i, carry):
        for j in range(FH // 16):
            zb[i, pl.ds(j * 16, 16)] = jnp.zeros((16,), _f32)
        return carry

    lax.fori_loop(0, ZROWS, zb_body, 0)

    crow0 = s * NCHT
    pltpu.sync_copy(src_hbm.at[pl.ds(crow0, NCHT)], idx_s2)
    pltpu.sync_copy(dst_hbm.at[pl.ds(crow0, NCHT)], idx_d2)
    pltpu.sync_copy(norm_hbm.at[pl.ds(crow0, NCHT)], nrm2)

    srcs = [(h0_hbm, h1_hbm), (o10, o11), (o20, o21)]
    outs = [(o10, o11), (o20, o21), (o30, o31)]
    for r in range(KHOPS):
        for k2 in range(RPT // ZROWS):
            pltpu.sync_copy(zb, acc.at[pl.ds(s * RPT + k2 * ZROWS, ZROWS)])
        plsc.subcore_barrier()

        g0, g1 = srcs[r]

        def start_gather(ci, b, g0=g0, g1=g1):
            @pl.when(c == 0)
            def _():
                pltpu.async_copy(g0.at[idx_s2.at[ci]], rows[b], gsem[b])

            @pl.when(c == 1)
            def _():
                pltpu.async_copy(g1.at[idx_s2.at[ci]], rows[b], gsem[b])

        def wait_gather(b, g0=g0):
            pltpu.make_async_copy(g0.at[idx_s2.at[0]], rows[b], gsem[b]).wait()

        def wait_scatter(b):
            pltpu.make_async_copy(rows[b], acc.at[idx_d2.at[0]], ssem[b]).wait()

        start_gather(0, 0)
        start_gather(1, 1)

        def group_body(gi, carry):
            c0 = gi * NRING
            for b in range(NRING):
                ci = c0 + b
                wait_gather(b)
                _scale_rows(rows[b], nrm2, ci)
                pltpu.async_copy(rows[b], acc.at[idx_d2.at[ci]], ssem[b],
                                 add=True)
                nb = (b + 2) % NRING
                nc = ci + 2
                if b < 2:
                    @pl.when(gi > 0)
                    def _():
                        wait_scatter(nb)
                else:
                    wait_scatter(nb)

                start_gather(nc, nb)
            return carry

        lax.fori_loop(0, NGRP, group_body, 0)

        for b in range(NEPI):
            ci = NGRP * NRING + b
            wait_gather(b)
            _scale_rows(rows[b], nrm2, ci)
            pltpu.async_copy(rows[b], acc.at[idx_d2.at[ci]], ssem[b], add=True)
        for b in range(NRING):
            wait_scatter(b)
        plsc.subcore_barrier()

        oo0, oo1 = outs[r]
        row0 = s * RPT

        @pl.when(c == 0)
        def _(oo0=oo0):
            pltpu.sync_copy(acc.at[pl.ds(row0, RPT)], oo0.at[pl.ds(row0, RPT)])

        @pl.when(c == 1)
        def _(oo1=oo1):
            pltpu.sync_copy(acc.at[pl.ds(row0, RPT)], oo1.at[pl.ds(row0, RPT)])

        plsc.subcore_barrier()


BM = 2000
_NBM = N // BM


def _mm_acc(parts, w_ref, b_ref):
    acc = b_ref[...]
    for i, p in enumerate(parts):
        acc = acc + jnp.dot(p[...], w_ref[i * FH:(i + 1) * FH, :],
                            preferred_element_type=_f32)
    return acc


def _mm_mid_body(p0, p1, p2, p3, p4, p5, p6, p7, w_ref, b_ref, y0, y1):
    acc = _mm_acc([p0, p1, p2, p3, p4, p5, p6, p7], w_ref, b_ref)
    acc = jnp.where(acc >= 0.0, acc, 0.01 * acc)
    y0[...] = acc[:, :FH]
    y1[...] = acc[:, FH:]


def _mm_out_body(p0, p1, p2, p3, p4, p5, p6, p7, w_ref, b_ref, m_ref, out):
    acc = _mm_acc([p0, p1, p2, p3, p4, p5, p6, p7], w_ref, b_ref)
    out[...] = acc * m_ref[...]


_part_spec = pl.BlockSpec((BM, FH), lambda i: (i, 0))
_w_spec = pl.BlockSpec(((KHOPS + 1) * F, F), lambda i: (0, 0))
_b_spec = pl.BlockSpec((1, F), lambda i: (0, 0))

_mm_mid = pl.pallas_call(
    _mm_mid_body,
    grid=(_NBM,),
    in_specs=[_part_spec] * 8 + [_w_spec, _b_spec],
    out_specs=[pl.BlockSpec((BM, FH), lambda i: (i, 0))] * 2,
    out_shape=[jax.ShapeDtypeStruct((N, FH), _f32)] * 2,
)

_mm_out = pl.pallas_call(
    _mm_out_body,
    grid=(_NBM,),
    in_specs=[_part_spec] * 8 + [_w_spec, _b_spec,
                                 pl.BlockSpec((BM, 1), lambda i: (i, 0))],
    out_specs=pl.BlockSpec((BM, F), lambda i: (i, 0)),
    out_shape=jax.ShapeDtypeStruct((N, F), _f32),
)


def kernel(x, edge_index, edge_weights, feature_mask, W0, b0, W1, b1):
    src = edge_index[0]
    dst = edge_index[1]
    norm = _prep(src, dst, edge_weights)
    src2 = src.reshape(E // CH, CH)
    dst2 = dst.reshape(E // CH, CH)
    norm2 = norm.reshape(E // CH, CH)

    x0 = x[:, :FH]
    x1 = x[:, FH:]
    h10, h11, h20, h21, h30, h31 = _layer(x0, x1, src2, dst2, norm2)
    y0, y1 = _mm_mid(x0, x1, h10, h11, h20, h21, h30, h31,
                     W0, b0.reshape(1, F))
    g10, g11, g20, g21, g30, g31 = _layer(y0, y1, src2, dst2, norm2)
    out = _mm_out(y0, y1, g10, g11, g20, g21, g30, g31,
                  W1, b1.reshape(1, F), feature_mask[:, None])
    return out


# speedup vs baseline: 10.8787x; 1.3109x over previous
import functools

import jax
import jax.numpy as jnp
from jax import lax
from jax.experimental import pallas as pl
from jax.experimental.pallas import tpu as pltpu
from jax.experimental.pallas import tpu_sc as plsc

N = 10000
E = 320000
F = 128
FH = 64
KHOPS = 3

NC = 2
NS = 16
CH = 80
EPT = E // NS
NCH = EPT // CH
EPW = E // (NC * NS)
NCHW = EPW // CH
NR = 10240
RPT = NR // NS
ZROWS = 128
DEG_PAD = 10240

_mesh = plsc.VectorSubcoreMesh(core_axis_name="c", subcore_axis_name="s")

_GATHER_1D = lax.GatherDimensionNumbers(
    offset_dims=(), collapsed_slice_dims=(0,), start_index_map=(0,))


def _bcast_lane(v16, lane):
    idx = jnp.full((16, 1), lane, _i32)
    return lax.gather(v16, idx, _GATHER_1D, (1,),
                      mode=lax.GatherScatterMode.PROMISE_IN_BOUNDS)

_f32 = jnp.float32
_i32 = jnp.int32


def _fast_rsqrt(v):
    i = lax.bitcast_convert_type(v, _i32)
    y = lax.bitcast_convert_type(jnp.int32(0x5F3759DF) - (i >> 1), _f32)
    for _ in range(3):
        y = y * (1.5 - 0.5 * v * y * y)
    return y


_NCR = E // CH
_TROWS = _NCR // NS
_WROWS = _NCR // (NC * NS)


@functools.partial(
    pl.kernel,
    out_type=jax.ShapeDtypeStruct((_NCR, CH), _f32),
    mesh=_mesh,
    scratch_types=[
        pltpu.VMEM((640,), _f32),
        pltpu.VMEM((_TROWS, CH), _i32),
        pltpu.VMEM((_TROWS, CH), _f32),
        pltpu.VMEM((_WROWS, CH), _i32),
        pltpu.VMEM((_WROWS, CH), _i32),
        pltpu.VMEM((_WROWS, CH), _f32),
        pltpu.VMEM((_WROWS, CH), _f32),
        pltpu.VMEM((N,), _f32),
        pltpu.VMEM_SHARED((DEG_PAD,), _f32),
    ],
    compiler_params=pltpu.CompilerParams(needs_layout_passes=False,
                                         use_tc_tiling_on_sc=False),
    name="tag_prep_norm",
)
def _prep(src_hbm, dst_hbm, ew_hbm, norm_hbm,
          zb, dstb, ewb, srcb2, dstb2, ewb2, nrmb2, dinv, deg_sh):
    c = lax.axis_index("c")
    s = lax.axis_index("s")

    def zb_body(i, carry):
        zb[pl.ds(i * 16, 16)] = jnp.zeros((16,), _f32)
        return carry

    lax.fori_loop(0, 640 // 16, zb_body, 0)
    pltpu.sync_copy(zb, deg_sh.at[pl.ds(s * 640, 640)])

    pltpu.sync_copy(dst_hbm.at[pl.ds(s * _TROWS, _TROWS)], dstb)
    pltpu.sync_copy(ew_hbm.at[pl.ds(s * _TROWS, _TROWS)], ewb)
    plsc.subcore_barrier()

    def deg_body(ci, carry):
        pltpu.sync_copy(ewb.at[ci], deg_sh.at[dstb.at[ci]], add=True)
        return carry

    lax.fori_loop(0, _TROWS, deg_body, 0)
    plsc.subcore_barrier()

    pltpu.sync_copy(deg_sh.at[pl.ds(0, N)], dinv)
    wid = c * NS + s
    pltpu.sync_copy(src_hbm.at[pl.ds(wid * _WROWS, _WROWS)], srcb2)
    pltpu.sync_copy(dst_hbm.at[pl.ds(wid * _WROWS, _WROWS)], dstb2)
    pltpu.sync_copy(ew_hbm.at[pl.ds(wid * _WROWS, _WROWS)], ewb2)

    def dinv_body(i, carry):
        v = dinv[pl.ds(i * 16, 16)]
        r = _fast_rsqrt(v)
        dinv[pl.ds(i * 16, 16)] = jnp.where(v > 0.0, r, 0.0)
        return carry

    lax.fori_loop(0, N // 16, dinv_body, 0)

    def norm_body(ci, carry):
        for g in range(CH // 16):
            sl = pl.ds(g * 16, 16)
            a = plsc.load_gather(dinv, [srcb2[ci, sl]])
            b = plsc.load_gather(dinv, [dstb2[ci, sl]])
            nrmb2[ci, sl] = a * ewb2[ci, sl] * b
        return carry

    lax.fori_loop(0, _WROWS, norm_body, 0)
    pltpu.sync_copy(nrmb2, norm_hbm.at[pl.ds(wid * _WROWS, _WROWS)])


NRING = 4
NGRP = NCH // NRING
NEPI = NCH - NGRP * NRING
NCHT = EPT // CH


def _scale_rows(rows, nrm2, ci):
    for g in range(CH // 16):
        n16 = nrm2[ci, pl.ds(g * 16, 16)]
        for lane in range(16):
            e = g * 16 + lane
            bb = _bcast_lane(n16, lane)
            for j in range(FH // 16):
                sl = pl.ds(j * 16, 16)
                rows[e, sl] = rows[e, sl] * bb


@functools.partial(
    pl.kernel,
    out_type=[jax.ShapeDtypeStruct((NR, FH), _f32) for _ in range(2 * KHOPS)],
    mesh=_mesh,
    scratch_types=[
        pltpu.VMEM((ZROWS, FH), _f32),
        pltpu.VMEM((NCHT, CH), _i32),
        pltpu.VMEM((NCHT, CH), _i32),
        pltpu.VMEM((NCHT, CH), _f32),
        [pltpu.VMEM((CH, FH), _f32) for _ in range(NRING)],
        [pltpu.SemaphoreType.DMA for _ in range(NRING)],
        [pltpu.SemaphoreType.DMA for _ in range(NRING)],
        pltpu.VMEM_SHARED((NR, FH), _f32),
    ],
    compiler_params=pltpu.CompilerParams(needs_layout_passes=False,
                                         use_tc_tiling_on_sc=False),
    name="tag_propagate",
)
def _layer(h0_hbm, h1_hbm, src_hbm, dst_hbm, norm_hbm,
           o10, o11, o20, o21, o30, o31,
           zb, idx_s2, idx_d2, nrm2, rows, gsem, ssem, acc):
    c = lax.axis_index("c")
    s = lax.axis_index("s")

    def zb_body(i, carry):
        for j in range(FH // 16):
            zb[i, pl.ds(j * 16, 16)] = jnp.zeros((16,), _f32)
        return carry

    lax.fori_loop(0, ZROWS, zb_body, 0)

    crow0 = s * NCHT
    pltpu.sync_copy(src_hbm.at[pl.ds(crow0, NCHT)], idx_s2)
    pltpu.sync_copy(dst_hbm.at[pl.ds(crow0, NCHT)], idx_d2)
    pltpu.sync_copy(norm_hbm.at[pl.ds(crow0, NCHT)], nrm2)

    srcs = [(h0_hbm, h1_hbm), (o10, o11), (o20, o21)]
    outs = [(o10, o11), (o20, o21), (o30, o31)]
    for r in range(KHOPS):
        for k2 in range(RPT // ZROWS):
            pltpu.sync_copy(zb, acc.at[pl.ds(s * RPT + k2 * ZROWS, ZROWS)])
        plsc.subcore_barrier()

        g0, g1 = srcs[r]

        def start_gather(ci, b, g0=g0, g1=g1):
            @pl.when(c == 0)
            def _():
                pltpu.async_copy(g0.at[idx_s2.at[ci]], rows[b], gsem[b])

            @pl.when(c == 1)
            def _():
                pltpu.async_copy(g1.at[idx_s2.at[ci]], rows[b], gsem[b])

        def wait_gather(b, g0=g0):
            pltpu.make_async_copy(g0.at[idx_s2.at[0]], rows[b], gsem[b]).wait()

        def wait_scatter(b):
            pltpu.make_async_copy(rows[b], acc.at[idx_d2.at[0]], ssem[b]).wait()

        start_gather(0, 0)
        start_gather(1, 1)

        def group_body(gi, carry):
            c0 = gi * NRING
            for b in range(NRING):
                ci = c0 + b
                wait_gather(b)
                _scale_rows(rows[b], nrm2, ci)
                pltpu.async_copy(rows[b], acc.at[idx_d2.at[ci]], ssem[b],
                                 add=True)
                nb = (b + 2) % NRING
                nc = ci + 2
                if b < 2:
                    @pl.when(gi > 0)
                    def _():
                        wait_scatter(nb)
                else:
                    wait_scatter(nb)

                start_gather(nc, nb)
            return carry

        lax.fori_loop(0, NGRP, group_body, 0)

        for b in range(NEPI):
            ci = NGRP * NRING + b
            wait_gather(b)
            _scale_rows(rows[b], nrm2, ci)
            pltpu.async_copy(rows[b], acc.at[idx_d2.at[ci]], ssem[b], add=True)
        for b in range(NRING):
            wait_scatter(b)
        plsc.subcore_barrier()

        oo0, oo1 = outs[r]
        row0 = s * RPT

        @pl.when(c == 0)
        def _(oo0=oo0):
            pltpu.sync_copy(acc.at[pl.ds(row0, RPT)], oo0.at[pl.ds(row0, RPT)])

        @pl.when(c == 1)
        def _(oo1=oo1):
            pltpu.sync_copy(acc.at[pl.ds(row0, RPT)], oo1.at[pl.ds(row0, RPT)])

        plsc.subcore_barrier()


BM = 2000
_NBM = N // BM


def _mm_acc(parts, w_ref, b_ref):
    acc = b_ref[...]
    for i, p in enumerate(parts):
        acc = acc + jnp.dot(p[...], w_ref[i * FH:(i + 1) * FH, :],
                            preferred_element_type=_f32)
    return acc


def _mm_mid_body(p0, p1, p2, p3, p4, p5, p6, p7, w_ref, b_ref, y0, y1):
    acc = _mm_acc([p0, p1, p2, p3, p4, p5, p6, p7], w_ref, b_ref)
    acc = jnp.where(acc >= 0.0, acc, 0.01 * acc)
    y0[...] = acc[:, :FH]
    y1[...] = acc[:, FH:]


def _mm_out_body(p0, p1, p2, p3, p4, p5, p6, p7, w_ref, b_ref, m_ref, out):
    acc = _mm_acc([p0, p1, p2, p3, p4, p5, p6, p7], w_ref, b_ref)
    out[...] = acc * m_ref[...]


_part_spec = pl.BlockSpec((BM, FH), lambda i: (i, 0))
_w_spec = pl.BlockSpec(((KHOPS + 1) * F, F), lambda i: (0, 0))
_b_spec = pl.BlockSpec((1, F), lambda i: (0, 0))

_mm_mid = pl.pallas_call(
    _mm_mid_body,
    grid=(_NBM,),
    in_specs=[_part_spec] * 8 + [_w_spec, _b_spec],
    out_specs=[pl.BlockSpec((BM, FH), lambda i: (i, 0))] * 2,
    out_shape=[jax.ShapeDtypeStruct((N, FH), _f32)] * 2,
)

_mm_out = pl.pallas_call(
    _mm_out_body,
    grid=(_NBM,),
    in_specs=[_part_spec] * 8 + [_w_spec, _b_spec,
                                 pl.BlockSpec((BM, 1), lambda i: (i, 0))],
    out_specs=pl.BlockSpec((BM, F), lambda i: (i, 0)),
    out_shape=jax.ShapeDtypeStruct((N, F), _f32),
)


def kernel(x, edge_index, edge_weights, feature_mask, W0, b0, W1, b1):
    src2 = edge_index[0].reshape(E // CH, CH)
    dst2 = edge_index[1].reshape(E // CH, CH)
    ew2 = edge_weights.reshape(E // CH, CH)
    norm2 = _prep(src2, dst2, ew2)

    x0 = x[:, :FH]
    x1 = x[:, FH:]
    h10, h11, h20, h21, h30, h31 = _layer(x0, x1, src2, dst2, norm2)
    y0, y1 = _mm_mid(x0, x1, h10, h11, h20, h21, h30, h31,
                     W0, b0.reshape(1, F))
    g10, g11, g20, g21, g30, g31 = _layer(y0, y1, src2, dst2, norm2)
    out = _mm_out(y0, y1, g10, g11, g20, g21, g30, g31,
                  W1, b1.reshape(1, F), feature_mask[:, None])
    return out


# speedup vs baseline: 13.6319x; 1.2531x over previous
import functools

import jax
import jax.numpy as jnp
from jax import lax
from jax.experimental import pallas as pl
from jax.experimental.pallas import tpu as pltpu
from jax.experimental.pallas import tpu_sc as plsc

N = 10000
E = 320000
F = 128
FH = 64
KHOPS = 3

NC = 2
NS = 16
CH = 80
EPT = E // NS
NCH = EPT // CH
EPW = E // (NC * NS)
NCHW = EPW // CH
NR = 10240
RPT = NR // NS
ZROWS = 128
DEG_PAD = 10240

_mesh = plsc.VectorSubcoreMesh(core_axis_name="c", subcore_axis_name="s")

_GATHER_1D = lax.GatherDimensionNumbers(
    offset_dims=(), collapsed_slice_dims=(0,), start_index_map=(0,))


def _bcast_lane(v16, lane):
    idx = jnp.full((16, 1), lane, _i32)
    return lax.gather(v16, idx, _GATHER_1D, (1,),
                      mode=lax.GatherScatterMode.PROMISE_IN_BOUNDS)

_f32 = jnp.float32
_i32 = jnp.int32


def _fast_rsqrt(v):
    i = lax.bitcast_convert_type(v, _i32)
    y = lax.bitcast_convert_type(jnp.int32(0x5F3759DF) - (i >> 1), _f32)
    for _ in range(3):
        y = y * (1.5 - 0.5 * v * y * y)
    return y


_NCR = E // CH
_TROWS = _NCR // NS
_WROWS = _NCR // (NC * NS)


@functools.partial(
    pl.kernel,
    out_type=jax.ShapeDtypeStruct((_NCR, CH), _f32),
    mesh=_mesh,
    scratch_types=[
        pltpu.VMEM((640,), _f32),
        pltpu.VMEM((_TROWS, CH), _i32),
        pltpu.VMEM((_TROWS, CH), _f32),
        pltpu.VMEM((_WROWS, CH), _i32),
        pltpu.VMEM((_WROWS, CH), _i32),
        pltpu.VMEM((_WROWS, CH), _f32),
        pltpu.VMEM((_WROWS, CH), _f32),
        pltpu.VMEM((N,), _f32),
        pltpu.VMEM_SHARED((DEG_PAD,), _f32),
    ],
    compiler_params=pltpu.CompilerParams(needs_layout_passes=False,
                                         use_tc_tiling_on_sc=False),
    name="tag_prep_norm",
)
def _prep(src_hbm, dst_hbm, ew_hbm, norm_hbm,
          zb, dstb, ewb, srcb2, dstb2, ewb2, nrmb2, dinv, deg_sh):
    c = lax.axis_index("c")
    s = lax.axis_index("s")

    def zb_body(i, carry):
        zb[pl.ds(i * 16, 16)] = jnp.zeros((16,), _f32)
        return carry

    lax.fori_loop(0, 640 // 16, zb_body, 0)
    pltpu.sync_copy(zb, deg_sh.at[pl.ds(s * 640, 640)])

    pltpu.sync_copy(dst_hbm.at[pl.ds(s * _TROWS, _TROWS)], dstb)
    pltpu.sync_copy(ew_hbm.at[pl.ds(s * _TROWS, _TROWS)], ewb)
    plsc.subcore_barrier()

    def deg_body(ci, carry):
        pltpu.sync_copy(ewb.at[ci], deg_sh.at[dstb.at[ci]], add=True)
        return carry

    lax.fori_loop(0, _TROWS, deg_body, 0)
    plsc.subcore_barrier()

    pltpu.sync_copy(deg_sh.at[pl.ds(0, N)], dinv)
    wid = c * NS + s
    pltpu.sync_copy(src_hbm.at[pl.ds(wid * _WROWS, _WROWS)], srcb2)
    pltpu.sync_copy(dst_hbm.at[pl.ds(wid * _WROWS, _WROWS)], dstb2)
    pltpu.sync_copy(ew_hbm.at[pl.ds(wid * _WROWS, _WROWS)], ewb2)

    def dinv_body(i, carry):
        v = dinv[pl.ds(i * 16, 16)]
        r = _fast_rsqrt(v)
        dinv[pl.ds(i * 16, 16)] = jnp.where(v > 0.0, r, 0.0)
        return carry

    lax.fori_loop(0, N // 16, dinv_body, 0)

    def norm_body(ci, carry):
        for g in range(CH // 16):
            sl = pl.ds(g * 16, 16)
            a = plsc.load_gather(dinv, [srcb2[ci, sl]])
            b = plsc.load_gather(dinv, [dstb2[ci, sl]])
            nrmb2[ci, sl] = a * ewb2[ci, sl] * b
        return carry

    lax.fori_loop(0, _WROWS, norm_body, 0)
    pltpu.sync_copy(nrmb2, norm_hbm.at[pl.ds(wid * _WROWS, _WROWS)])


NRING = 3
NGRP = NCH // NRING
NEPI = NCH - NGRP * NRING
NCHT = EPT // CH
CVR = 16

_bf16 = jnp.bfloat16
_M16 = -65536
_RND = 32768
FHW = FH // 2


def _unpack_scale(bfb, frows, nrm2, ci):
    for g in range(CH // 16):
        n16 = nrm2[ci, pl.ds(g * 16, 16)]
        for lane in range(16):
            e = g * 16 + lane
            bb = _bcast_lane(n16, lane)
            for w in range(2):
                v = bfb[e, pl.ds(16 * w, 16)]
                lo = lax.bitcast_convert_type(v << 16, _f32)
                hi = lax.bitcast_convert_type(v & _M16, _f32)
                frows[e, pl.ds(16 * w, 16)] = lo * bb
                frows[e, pl.ds(32 + 16 * w, 16)] = hi * bb


@functools.partial(
    pl.kernel,
    out_type=[jax.ShapeDtypeStruct((NR, FHW), _i32) for _ in range(2 * KHOPS)],
    mesh=_mesh,
    scratch_types=[
        pltpu.VMEM((NCHT, CH), _i32),
        pltpu.VMEM((NCHT, CH), _i32),
        pltpu.VMEM((NCHT, CH), _f32),
        [pltpu.VMEM((CH, FHW), _i32) for _ in range(NRING)],
        [pltpu.VMEM((CH, FH), _f32) for _ in range(NRING)],
        [pltpu.SemaphoreType.DMA for _ in range(NRING)],
        [pltpu.SemaphoreType.DMA for _ in range(NRING)],
        pltpu.VMEM_SHARED((NR, FH), _f32),
    ],
    compiler_params=pltpu.CompilerParams(needs_layout_passes=False,
                                         use_tc_tiling_on_sc=False),
    name="tag_propagate",
)
def _layer(h0_hbm, h1_hbm, src_hbm, dst_hbm, norm_hbm,
           o10, o11, o20, o21, o30, o31,
           idx_s2, idx_d2, nrm2, bfb, frows, gsem, ssem, acc):
    c = lax.axis_index("c")
    s = lax.axis_index("s")

    crow0 = s * NCHT
    pltpu.sync_copy(src_hbm.at[pl.ds(crow0, NCHT)], idx_s2)
    pltpu.sync_copy(dst_hbm.at[pl.ds(crow0, NCHT)], idx_d2)
    pltpu.sync_copy(norm_hbm.at[pl.ds(crow0, NCHT)], nrm2)

    srcs = [(h0_hbm, h1_hbm), (o10, o11), (o20, o21)]
    outs = [(o10, o11), (o20, o21), (o30, o31)]
    for r in range(KHOPS):
        def zf_body(i, carry):
            for j in range(FH // 16):
                frows[0][i, pl.ds(j * 16, 16)] = jnp.zeros((16,), _f32)
            return carry

        lax.fori_loop(0, CH, zf_body, 0)
        for k2 in range(RPT // CH):
            pltpu.sync_copy(frows[0], acc.at[pl.ds(s * RPT + k2 * CH, CH)])
        plsc.subcore_barrier()

        g0, g1 = srcs[r]

        def start_gather(ci, b, g0=g0, g1=g1):
            @pl.when(c == 0)
            def _():
                pltpu.async_copy(g0.at[idx_s2.at[ci]], bfb[b], gsem[b])

            @pl.when(c == 1)
            def _():
                pltpu.async_copy(g1.at[idx_s2.at[ci]], bfb[b], gsem[b])

        def wait_gather(b, g0=g0):
            pltpu.make_async_copy(g0.at[idx_s2.at[0]], bfb[b], gsem[b]).wait()

        def wait_scatter(b):
            pltpu.make_async_copy(frows[b], acc.at[idx_d2.at[0]],
                                  ssem[b]).wait()

        start_gather(0, 0)
        start_gather(1, 1)

        def group_body(gi, carry):
            c0 = gi * NRING
            for b in range(NRING):
                ci = c0 + b
                wait_gather(b)
                @pl.when(gi > 0)
                def _():
                    wait_scatter(b)

                _unpack_scale(bfb[b], frows[b], nrm2, ci)
                pltpu.async_copy(frows[b], acc.at[idx_d2.at[ci]], ssem[b],
                                 add=True)
                nc = ci + 2
                nb = (b + 2) % NRING

                @pl.when(nc < NCH)
                def _(nc=nc, nb=nb):
                    start_gather(nc, nb)
            return carry

        lax.fori_loop(0, NGRP, group_body, 0)

        for b in range(NEPI):
            ci = NGRP * NRING + b
            wait_gather(b)
            wait_scatter(b)
            _unpack_scale(bfb[b], frows[b], nrm2, ci)
            pltpu.async_copy(frows[b], acc.at[idx_d2.at[ci]], ssem[b],
                             add=True)
        for b in range(NRING):
            wait_scatter(b)
        plsc.subcore_barrier()

        oo0, oo1 = outs[r]
        row0 = s * RPT
        stg = frows[0].at[pl.ds(0, CVR)]
        bstg = bfb[0].at[pl.ds(0, CVR)]

        def conv_body(k, carry, oo0=oo0, oo1=oo1):
            r0 = row0 + k * CVR
            pltpu.sync_copy(acc.at[pl.ds(r0, CVR)], stg)
            for e in range(CVR):
                for w in range(2):
                    a = lax.bitcast_convert_type(
                        frows[0][e, pl.ds(16 * w, 16)], _i32)
                    b_ = lax.bitcast_convert_type(
                        frows[0][e, pl.ds(32 + 16 * w, 16)], _i32)
                    word = ((((a + _RND) >> 16) & 0xFFFF)
                            | ((b_ + _RND) & _M16))
                    bfb[0][e, pl.ds(16 * w, 16)] = word

            @pl.when(c == 0)
            def _():
                pltpu.sync_copy(bstg, oo0.at[pl.ds(r0, CVR)])

            @pl.when(c == 1)
            def _():
                pltpu.sync_copy(bstg, oo1.at[pl.ds(r0, CVR)])

            return carry

        lax.fori_loop(0, RPT // CVR, conv_body, 0)
        plsc.subcore_barrier()


BM = 2000
_NBM = N // BM


def _unpack_cols(v):
    lo = lax.bitcast_convert_type(v << 16, _f32)
    hi = lax.bitcast_convert_type(v & _M16, _f32)
    return lo, hi


def _pack_cols(a, b):
    ai = lax.bitcast_convert_type(a, _i32)
    bi = lax.bitcast_convert_type(b, _i32)
    return (((ai + _RND) >> 16) & 0xFFFF) | ((bi + _RND) & _M16)


def _packed_acc(acc, parts, w_ref, base):
    for i, p in enumerate(parts):
        r0 = base + i * FH
        lo, hi = _unpack_cols(p[...])
        acc = acc + jnp.dot(lo, w_ref[r0:r0 + FHW, :],
                            preferred_element_type=_f32)
        acc = acc + jnp.dot(hi, w_ref[r0 + FHW:r0 + FH, :],
                            preferred_element_type=_f32)
    return acc


def _mm_mid_body(x0, x1, p2, p3, p4, p5, p6, p7, w_ref, b_ref,
                 y0p, y1p, yf):
    acc = b_ref[...]
    acc = acc + jnp.dot(x0[...], w_ref[0:FH, :], preferred_element_type=_f32)
    acc = acc + jnp.dot(x1[...], w_ref[FH:2 * FH, :],
                        preferred_element_type=_f32)
    acc = _packed_acc(acc, [p2, p3, p4, p5, p6, p7], w_ref, 2 * FH)
    acc = jnp.where(acc >= 0.0, acc, 0.01 * acc)
    yf[...] = acc
    y0p[...] = _pack_cols(acc[:, 0:FHW], acc[:, FHW:FH])
    y1p[...] = _pack_cols(acc[:, FH:FH + FHW], acc[:, FH + FHW:2 * FH])


def _mm_out_body(yf, p1, p2, p3, p4, p5, p6, w_ref, b_ref, m_ref, out):
    acc = b_ref[...] + jnp.dot(yf[...], w_ref[0:F, :],
                               preferred_element_type=_f32)
    acc = _packed_acc(acc, [p1, p2, p3, p4, p5, p6], w_ref, F)
    out[...] = acc * m_ref[...]


_xpart_spec = pl.BlockSpec((BM, FH), lambda i: (i, 0))
_ppart_spec = pl.BlockSpec((BM, FHW), lambda i: (i, 0))
_w_spec = pl.BlockSpec(((KHOPS + 1) * F, F), lambda i: (0, 0))
_b_spec = pl.BlockSpec((1, F), lambda i: (0, 0))

_mm_mid = pl.pallas_call(
    _mm_mid_body,
    grid=(_NBM,),
    in_specs=[_xpart_spec] * 2 + [_ppart_spec] * 6 + [_w_spec, _b_spec],
    out_specs=[pl.BlockSpec((BM, FHW), lambda i: (i, 0))] * 2
    + [pl.BlockSpec((BM, F), lambda i: (i, 0))],
    out_shape=[jax.ShapeDtypeStruct((N, FHW), _i32)] * 2
    + [jax.ShapeDtypeStruct((N, F), _f32)],
)

_mm_out = pl.pallas_call(
    _mm_out_body,
    grid=(_NBM,),
    in_specs=[pl.BlockSpec((BM, F), lambda i: (i, 0))] + [_ppart_spec] * 6
    + [_w_spec, _b_spec, pl.BlockSpec((BM, 1), lambda i: (i, 0))],
    out_specs=pl.BlockSpec((BM, F), lambda i: (i, 0)),
    out_shape=jax.ShapeDtypeStruct((N, F), _f32),
)


def kernel(x, edge_index, edge_weights, feature_mask, W0, b0, W1, b1):
    src2 = edge_index[0].reshape(E // CH, CH)
    dst2 = edge_index[1].reshape(E // CH, CH)
    ew2 = edge_weights.reshape(E // CH, CH)
    norm2 = _prep(src2, dst2, ew2)

    xb = lax.bitcast_convert_type(x.astype(_bf16), jnp.uint16).astype(_i32)
    x0p = xb[:, 0:FHW] | (xb[:, FHW:FH] << 16)
    x1p = xb[:, FH:FH + FHW] | (xb[:, FH + FHW:2 * FH] << 16)
    x0f = x[:, :FH]
    x1f = x[:, FH:]
    h10, h11, h20, h21, h30, h31 = _layer(x0p, x1p, src2, dst2, norm2)
    y0p, y1p, yf = _mm_mid(x0f, x1f, h10, h11, h20, h21, h30, h31,
                           W0, b0.reshape(1, F))
    g10, g11, g20, g21, g30, g31 = _layer(y0p, y1p, src2, dst2, norm2)
    out = _mm_out(yf, g10, g11, g20, g21, g30, g31,
                  W1, b1.reshape(1, F), feature_mask[:, None])
    return out
